# baseline (speedup 1.0000x reference)
import jax
import jax.numpy as jnp
from jax import lax
from jax.experimental import pallas as pl
from jax.experimental.pallas import tpu as pltpu
from jax.experimental.pallas import tpu_sc as plsc

N = 10000
E = 320000
H = 128
C = 10
G = 16
NP = 10240
ROWS = NP // 128
NC, NS = 2, 16
NW = NC * NS
CHUNK = 128
CPW = 80
PADC = NW * CPW
STRIPE = NP // NS
F32 = jnp.float32
I32 = jnp.int32


def _sc_mesh():
    return plsc.VectorSubcoreMesh(
        core_axis_name="c", subcore_axis_name="s", num_cores=NC, num_subcores=NS
    )


def _spmm_body(src2d, dst2d, y_hbm, zeros_hbm, out_hbm, sidx, didx, rows, acc,
               gsem, ssem, isem):
    c = lax.axis_index("c")
    s = lax.axis_index("s")
    wid = s * NC + c
    base = wid * CPW
    pltpu.sync_copy(src2d.at[pl.ds(base, 8)], sidx.at[0])
    pltpu.sync_copy(dst2d.at[pl.ds(base, 8)], didx.at[0])
    pltpu.sync_copy(zeros_hbm, acc.at[pl.ds(s * STRIPE, STRIPE)])
    plsc.subcore_barrier()

    cps_g = [None, None]
    cps_s = [None, None]
    ipf = [None, None, None, None]
    cps_g[0] = pltpu.async_copy(y_hbm.at[sidx.at[0, 0]], rows.at[0], gsem)
    for j in range(CPW):
        q = j & 1
        qn = (j + 1) & 1
        blk = j >> 3
        off = j & 7
        if off == 0:
            if blk > 0:
                ipf[2].wait()
                ipf[3].wait()
            if blk + 1 < CPW // 8:
                nslot = (blk + 1) & 1
                ipf[0] = pltpu.async_copy(
                    src2d.at[pl.ds(base + (blk + 1) * 8, 8)], sidx.at[nslot],
                    isem)
                ipf[1] = pltpu.async_copy(
                    dst2d.at[pl.ds(base + (blk + 1) * 8, 8)], didx.at[nslot],
                    isem)
        if off == 7:
            ipf[2], ipf[3] = ipf[0], ipf[1]
        if j >= 1:
            cps_s[qn].wait()
        if j + 1 < CPW:
            j1 = j + 1
            cps_g[qn] = pltpu.async_copy(
                y_hbm.at[sidx.at[(j1 >> 3) & 1, j1 & 7]], rows.at[qn], gsem)
        cps_g[q].wait()
        cps_s[q] = pltpu.async_copy(
            rows.at[q], acc.at[didx.at[blk & 1, off]], ssem, add=True)
    cps_s[(CPW - 1) & 1].wait()

    plsc.subcore_barrier()
    pltpu.sync_copy(acc.at[pl.ds(s * STRIPE, STRIPE)],
                    out_hbm.at[c, pl.ds(s * STRIPE, STRIPE)])


def _sc_spmm(src2d, dst2d, y):
    zeros = jnp.zeros((STRIPE, 128), F32)
    return pl.kernel(
        _spmm_body,
        out_type=jax.ShapeDtypeStruct((NC, NP, 128), F32),
        mesh=_sc_mesh(),
        scratch_types=[
            pltpu.VMEM((2, 8, CHUNK), I32),
            pltpu.VMEM((2, 8, CHUNK), I32),
            pltpu.VMEM((2, CHUNK, 128), F32),
            pltpu.VMEM_SHARED((NP, 128), F32),
            pltpu.SemaphoreType.DMA,
            pltpu.SemaphoreType.DMA,
            pltpu.SemaphoreType.DMA,
        ],
    )(src2d, dst2d, y, zeros)


def _merge_tile_hists(s, c, lhist, slots, tbuf, out_r):
    pltpu.sync_copy(lhist, slots.at[s])
    plsc.subcore_barrier()
    for t in range(NS):
        pltpu.sync_copy(slots.at[t, pl.ds(s * STRIPE, STRIPE)], tbuf.at[t])

    def red(i, _):
        acc = tbuf[0, pl.ds(i * 16, 16)]
        for t in range(1, NS):
            acc = acc + tbuf[t, pl.ds(i * 16, 16)]
        lhist[pl.ds(i * 16, 16)] = acc
        return 0

    lax.fori_loop(0, STRIPE // 16, red, 0)
    pltpu.sync_copy(lhist.at[pl.ds(0, STRIPE)],
                    out_r.at[c, pl.ds(s * STRIPE, STRIPE)])


def _zero_vec(ref, nwords):
    def z(i, _):
        ref[pl.ds(i * 16, 16)] = jnp.zeros((16,), F32)
        return 0

    lax.fori_loop(0, nwords // 16, z, 0)


def _sc_degree(dst2d):
    def body(dst2d_r, out_r, didx, lhist, tbuf, slots):
        c = lax.axis_index("c")
        s = lax.axis_index("s")
        wid = s * NC + c
        pltpu.sync_copy(dst2d_r.at[pl.ds(wid * CPW, CPW)], didx)
        _zero_vec(lhist, NP)
        ones16 = jnp.ones((16,), F32)
        for j in range(CPW):
            for k in range(8):
                iv = didx[j, pl.ds(k * 16, 16)]
                plsc.addupdate_scatter(lhist, [iv], ones16)
        _merge_tile_hists(s, c, lhist, slots, tbuf, out_r)

    return pl.kernel(
        body,
        out_type=jax.ShapeDtypeStruct((NC, NP), F32),
        mesh=_sc_mesh(),
        compiler_params=pltpu.CompilerParams(needs_layout_passes=False),
        scratch_types=[
            pltpu.VMEM((CPW, CHUNK), I32),
            pltpu.VMEM((NP,), F32),
            pltpu.VMEM((NS, STRIPE), F32),
            pltpu.VMEM_SHARED((NS, NP), F32),
        ],
    )(dst2d)


def _sc_keepsum(src2d, dst2d, keep1d):
    def body(src2d_r, dst2d_r, keep_r, out_r, sidx, didx, lkeep, lhist, tbuf,
             slots):
        c = lax.axis_index("c")
        s = lax.axis_index("s")
        wid = s * NC + c
        pltpu.sync_copy(src2d_r.at[pl.ds(wid * CPW, CPW)], sidx)
        pltpu.sync_copy(dst2d_r.at[pl.ds(wid * CPW, CPW)], didx)
        pltpu.sync_copy(keep_r, lkeep)
        _zero_vec(lhist, NP)
        for j in range(CPW):
            for k in range(8):
                siv = sidx[j, pl.ds(k * 16, 16)]
                kv = plsc.load_gather(lkeep, [siv])
                div = didx[j, pl.ds(k * 16, 16)]
                plsc.addupdate_scatter(lhist, [div], kv)
        _merge_tile_hists(s, c, lhist, slots, tbuf, out_r)

    return pl.kernel(
        body,
        out_type=jax.ShapeDtypeStruct((NC, NP), F32),
        mesh=_sc_mesh(),
        compiler_params=pltpu.CompilerParams(needs_layout_passes=False),
        scratch_types=[
            pltpu.VMEM((CPW, CHUNK), I32),
            pltpu.VMEM((CPW, CHUNK), I32),
            pltpu.VMEM((NP,), F32),
            pltpu.VMEM((NP,), F32),
            pltpu.VMEM((NS, STRIPE), F32),
            pltpu.VMEM_SHARED((NS, NP), F32),
        ],
    )(src2d, dst2d, keep1d)


def _tc_mm_body(a_ref, b_ref, o_ref):
    o_ref[...] = jnp.dot(a_ref[...], b_ref[...], preferred_element_type=F32)


def _tc_mm(a, b):
    return pl.pallas_call(
        _tc_mm_body,
        out_shape=jax.ShapeDtypeStruct((a.shape[0], b.shape[1]), F32),
    )(a, b)


def _tc_gate_mm_body(h_ref, gate_ref, w_ref, o_ref):
    o_ref[...] = jnp.dot(h_ref[...] * gate_ref[...], w_ref[...],
                         preferred_element_type=F32)


def _tc_gate_mm(h, gate_col, W2):
    return pl.pallas_call(
        _tc_gate_mm_body,
        out_shape=jax.ShapeDtypeStruct((NP, H), F32),
    )(h, gate_col, W2)


def _tc_pre_body(x_ref, w1_ref, b1_ref, hist_ref, y1_ref, self1_ref, dis1_ref):
    xw = jnp.dot(x_ref[...], w1_ref[...], preferred_element_type=F32)
    indeg = hist_ref[0] + hist_ref[1]
    dis = lax.rsqrt(indeg + 1.0)
    y1_ref[...] = xw * dis
    self1_ref[...] = xw * (dis * dis) + b1_ref[...]
    dis1_ref[...] = dis


def _tc_pre(x, W1, b1row, hist_col):
    return pl.pallas_call(
        _tc_pre_body,
        out_shape=(
            jax.ShapeDtypeStruct((NP, H), F32),
            jax.ShapeDtypeStruct((NP, H), F32),
            jax.ShapeDtypeStruct((NP, 1), F32),
        ),
    )(x, W1, b1row, hist_col)


def _tc_h_topk_body(agg_ref, dis1_ref, self1_ref, p_ref, batch_ref,
                    h_ref, keep_ref, gate_ref):
    aggsum = agg_ref[0] + agg_ref[1]
    h = jnp.maximum(aggsum * dis1_ref[...] + self1_ref[...], 0.0)
    h_ref[...] = h
    p = p_ref[...]
    nrm = jnp.sqrt(jnp.sum(p * p))
    score0 = jnp.tanh(jnp.sum(h * p[None, None, :], axis=2) / nrm)
    score = score0
    score = jnp.where(score == 0.0, 0.0, score)
    bits = lax.bitcast_convert_type(score, I32)
    key = bits ^ ((bits >> 31) & jnp.int32(0x7FFFFFFF))
    batch = batch_ref[...]

    masks = [batch == g for g in range(G)]
    cnt = [jnp.sum(jnp.where(masks[g], 1.0, 0.0)) for g in range(G)]
    kf = [jnp.floor((cnt[g] + 1.0) * 0.5) for g in range(G)]

    lo0 = jnp.int32(-1065353218)
    hi0 = jnp.int32(1065353217)

    def body(_, carry):
        los, his = carry[:G], carry[G:]
        nlo, nhi = [], []
        for g in range(G):
            lo, hi = los[g], his[g]
            mid = lo + (hi - lo + 1) // 2
            cg = jnp.sum(jnp.where(masks[g] & (key >= mid), 1.0, 0.0))
            ok = cg >= kf[g]
            nlo.append(jnp.where(ok, mid, lo))
            nhi.append(jnp.where(ok, hi, mid - jnp.int32(1)))
        return tuple(nlo) + tuple(nhi)

    init = tuple([lo0] * G) + tuple([hi0] * G)
    res = lax.fori_loop(0, 32, body, init)
    v = res[:G]

    vbc = jnp.full(key.shape, jnp.int32(-2147483648))
    for g in range(G):
        vbc = jnp.where(masks[g], v[g], vbc)
    gt = key > vbc
    tie = key == vbc

    needbc = jnp.full(key.shape, -1.0)
    sbc = jnp.zeros(key.shape, F32)
    s_run = jnp.float32(0.0)
    for g in range(G):
        cnt_gt = jnp.sum(jnp.where(masks[g] & gt, 1.0, 0.0))
        needbc = jnp.where(masks[g], kf[g] - cnt_gt, needbc)
        sbc = jnp.where(masks[g], s_run, sbc)
        s_run = s_run + jnp.sum(jnp.where(masks[g] & tie, 1.0, 0.0))

    tie_f = jnp.where(tie, 1.0, 0.0)
    r1 = lax.broadcasted_iota(I32, (128, 128), 0)
    c1 = lax.broadcasted_iota(I32, (128, 128), 1)
    u_incl = jnp.where(r1 <= c1, 1.0, 0.0)
    pc = jnp.dot(tie_f, u_incl, preferred_element_type=F32)
    rt = jnp.dot(tie_f, jnp.ones((128, 1), F32), preferred_element_type=F32)
    r2 = lax.broadcasted_iota(I32, (ROWS, ROWS), 0)
    c2 = lax.broadcasted_iota(I32, (ROWS, ROWS), 1)
    l_strict = jnp.where(r2 > c2, 1.0, 0.0)
    row_off = jnp.dot(l_strict, rt, preferred_element_type=F32)
    excl = pc - tie_f + row_off
    tie_rank = excl - sbc

    keep = jnp.where(gt | (tie & (tie_rank < needbc)), 1.0, 0.0)
    keep_ref[...] = keep
    gate_ref[...] = score0 * keep


def _tc_h_topk(agg1_3, dis1_3, self1_3, p, batch2d):
    return pl.pallas_call(
        _tc_h_topk_body,
        out_shape=(
            jax.ShapeDtypeStruct((ROWS, 128, H), F32),
            jax.ShapeDtypeStruct((ROWS, 128), F32),
            jax.ShapeDtypeStruct((ROWS, 128), F32),
        ),
    )(agg1_3, dis1_3, self1_3, p, batch2d)


def _tc_mid_body(h_ref, gate_ref, keep_ref, w2_ref, b2_ref, ks_ref,
                 y2_ref, self2_ref, d2k_ref):
    xw2 = jnp.dot(h_ref[...] * gate_ref[...], w2_ref[...],
                  preferred_element_type=F32)
    keep = keep_ref[...]
    ks = ks_ref[0] + ks_ref[1]
    deg2 = keep * (ks + 1.0)
    deg2 = jnp.where(deg2 > 0.0, deg2, 1.0)
    dis2 = lax.rsqrt(deg2)
    y2_ref[...] = xw2 * dis2
    self2_ref[...] = xw2 * (dis2 * dis2) + b2_ref[...] * keep
    d2k_ref[...] = dis2 * keep


def _tc_mid(h, gate_col, keep_col, W2, b2row, ks_col):
    return pl.pallas_call(
        _tc_mid_body,
        out_shape=(
            jax.ShapeDtypeStruct((NP, H), F32),
            jax.ShapeDtypeStruct((NP, H), F32),
            jax.ShapeDtypeStruct((NP, 1), F32),
        ),
    )(h, gate_col, keep_col, W2, b2row, ks_col)


def _tc_post_body(agg_ref, d2k_ref, self2_ref, keep_ref, batch_ref, lw_ref,
                  lb_ref, out_ref):
    aggsum = agg_ref[0] + agg_ref[1]
    h2 = jnp.maximum(aggsum * d2k_ref[...] + self2_ref[...], 0.0)
    iota_g = lax.broadcasted_iota(I32, (G, NP), 0)
    oh = jnp.where(iota_g == batch_ref[...], 1.0, 0.0)
    summ = jnp.dot(oh, h2, preferred_element_type=F32)
    cnt = jnp.dot(oh, keep_ref[...], preferred_element_type=F32)
    mean = summ / jnp.maximum(cnt, 1.0)
    out_ref[...] = jnp.dot(mean, lw_ref[...], preferred_element_type=F32) \
        + lb_ref[...]


def _tc_post(agg2, d2k_col, self2, keep_col, batch_row, linW, linbrow):
    return pl.pallas_call(
        _tc_post_body,
        out_shape=jax.ShapeDtypeStruct((G, C), F32),
    )(agg2, d2k_col, self2, keep_col, batch_row, linW, linbrow)


def kernel(x, edge_index, batch, W1, b1, p, W2, b2, linW, linb):
    padlen = PADC * CHUNK - E
    pad_idx = (N + (jnp.arange(padlen) % (NP - N))).astype(I32)
    src2d = jnp.concatenate([edge_index[0], pad_idx]).reshape(PADC, CHUNK)
    dst2d = jnp.concatenate([edge_index[1], pad_idx]).reshape(PADC, CHUNK)
    x_p = jnp.pad(x, ((0, NP - N), (0, 0)))
    batch_p = jnp.pad(batch, (0, NP - N), constant_values=G)
    batch2d = batch_p.reshape(ROWS, 128)
    batch_row = batch_p.reshape(1, NP)

    hist = _sc_degree(dst2d)
    hist_col = hist.reshape(NC, NP, 1)
    y1, self1, dis1_col = _tc_pre(x_p, W1, b1.reshape(1, H), hist_col)

    agg1 = _sc_spmm(src2d, dst2d, y1)
    h3, keep2d, gate2d = _tc_h_topk(
        agg1.reshape(NC, ROWS, 128, H), dis1_col.reshape(ROWS, 128, 1),
        self1.reshape(ROWS, 128, H), p, batch2d)
    h = h3.reshape(NP, H)
    keep1d = keep2d.reshape(NP)
    keep_col = keep1d.reshape(NP, 1)
    gate_col = gate2d.reshape(NP, 1)

    ks = _sc_keepsum(src2d, dst2d, keep1d)
    ks_col = ks.reshape(NC, NP, 1)
    y2, self2, d2k_col = _tc_mid(h, gate_col, keep_col, W2,
                                 b2.reshape(1, H), ks_col)

    agg2 = _sc_spmm(src2d, dst2d, y2)
    return _tc_post(agg2, d2k_col, self2, keep_col, batch_row, linW,
                    linb.reshape(1, C))


# speedup vs baseline: 2.1658x; 2.1658x over previous
import jax
import jax.numpy as jnp
from jax import lax
from jax.experimental import pallas as pl
from jax.experimental.pallas import tpu as pltpu
from jax.experimental.pallas import tpu_sc as plsc

N = 10000
E = 320000
H = 128
C = 10
G = 16
NP = 10240
ROWS = NP // 128
NC, NS = 2, 16
NW = NC * NS
CHUNK = 128
CPW = 80
PADC = NW * CPW
STRIPE = NP // NS
SCH = 100
SCPW = 104
SNCH = NW * SCPW
F32 = jnp.float32
I32 = jnp.int32


def _sc_mesh():
    return plsc.VectorSubcoreMesh(
        core_axis_name="c", subcore_axis_name="s", num_cores=NC, num_subcores=NS
    )


def _spmm_body(src2d, dst2d, y_hbm, zeros_hbm, out_hbm, sidx, didx, rows, acc,
               gsem, ssem, isem):
    c = lax.axis_index("c")
    s = lax.axis_index("s")
    wid = s * NC + c
    base = wid * SCPW
    nblk = SCPW // 8
    pltpu.sync_copy(src2d.at[pl.ds(base, 8)], sidx.at[0])
    pltpu.sync_copy(dst2d.at[pl.ds(base, 8)], didx.at[0])
    pltpu.sync_copy(zeros_hbm, acc.at[pl.ds(s * STRIPE, STRIPE)])
    plsc.subcore_barrier()

    cps_g = [None, None, None]
    cps_s = [None, None, None]
    ipf = [None, None, None, None]
    cps_g[0] = pltpu.async_copy(y_hbm.at[sidx.at[0, 0]], rows.at[0], gsem)
    cps_g[1] = pltpu.async_copy(y_hbm.at[sidx.at[0, 1]], rows.at[1], gsem)
    for j in range(SCPW):
        q = j % 3
        qn = (j + 2) % 3
        blk = j >> 3
        off = j & 7
        if off == 0:
            if blk > 0:
                ipf[2].wait()
                ipf[3].wait()
            if blk + 1 < nblk:
                nslot = (blk + 1) & 1
                ipf[0] = pltpu.async_copy(
                    src2d.at[pl.ds(base + (blk + 1) * 8, 8)],
                    sidx.at[nslot], isem)
                ipf[1] = pltpu.async_copy(
                    dst2d.at[pl.ds(base + (blk + 1) * 8, 8)],
                    didx.at[nslot], isem)
        if off == 7:
            ipf[2], ipf[3] = ipf[0], ipf[1]
        if j >= 1:
            cps_s[qn].wait()
        if j + 2 < SCPW:
            j2 = j + 2
            cps_g[qn] = pltpu.async_copy(
                y_hbm.at[sidx.at[(j2 >> 3) & 1, j2 & 7]], rows.at[qn], gsem)
        cps_g[q].wait()
        cps_s[q] = pltpu.async_copy(
            rows.at[q], acc.at[didx.at[blk & 1, off]], ssem, add=True)
    cps_s[(SCPW - 1) % 3].wait()

    plsc.subcore_barrier()
    pltpu.sync_copy(acc.at[pl.ds(s * STRIPE, STRIPE)],
                    out_hbm.at[c, pl.ds(s * STRIPE, STRIPE)])


def _sc_spmm(src2d, dst2d, y):
    zeros = jnp.zeros((STRIPE, 128), F32)
    return pl.kernel(
        _spmm_body,
        out_type=jax.ShapeDtypeStruct((NC, NP, 128), F32),
        mesh=_sc_mesh(),
        scratch_types=[
            pltpu.VMEM((2, 8, SCH), I32),
            pltpu.VMEM((2, 8, SCH), I32),
            pltpu.VMEM((3, SCH, 128), F32),
            pltpu.VMEM_SHARED((NP, 128), F32),
            pltpu.SemaphoreType.DMA,
            pltpu.SemaphoreType.DMA,
            pltpu.SemaphoreType.DMA,
        ],
    )(src2d, dst2d, y, zeros)


def _merge_tile_hists(s, c, lhist, slots, tbuf, out_r):
    pltpu.sync_copy(lhist, slots.at[s])
    plsc.subcore_barrier()
    for t in range(NS):
        pltpu.sync_copy(slots.at[t, pl.ds(s * STRIPE, STRIPE)], tbuf.at[t])

    def red(i, _):
        acc = tbuf[0, pl.ds(i * 16, 16)]
        for t in range(1, NS):
            acc = acc + tbuf[t, pl.ds(i * 16, 16)]
        lhist[pl.ds(i * 16, 16)] = acc
        return 0

    lax.fori_loop(0, STRIPE // 16, red, 0)
    pltpu.sync_copy(lhist.at[pl.ds(0, STRIPE)],
                    out_r.at[c, pl.ds(s * STRIPE, STRIPE)])


def _zero_vec(ref, nwords):
    def z(i, _):
        ref[pl.ds(i * 16, 16)] = jnp.zeros((16,), F32)
        return 0

    lax.fori_loop(0, nwords // 16, z, 0)


def _sc_degree(dst2d):
    def body(dst2d_r, out_r, didx, lhist, tbuf, slots):
        c = lax.axis_index("c")
        s = lax.axis_index("s")
        wid = s * NC + c
        pltpu.sync_copy(dst2d_r.at[pl.ds(wid * CPW, CPW)], didx)
        _zero_vec(lhist, NP)
        ones16 = jnp.ones((16,), F32)
        for j in range(CPW):
            for k in range(8):
                iv = didx[j, pl.ds(k * 16, 16)]
                plsc.addupdate_scatter(lhist, [iv], ones16)
        _merge_tile_hists(s, c, lhist, slots, tbuf, out_r)

    return pl.kernel(
        body,
        out_type=jax.ShapeDtypeStruct((NC, NP), F32),
        mesh=_sc_mesh(),
        compiler_params=pltpu.CompilerParams(needs_layout_passes=False),
        scratch_types=[
            pltpu.VMEM((CPW, CHUNK), I32),
            pltpu.VMEM((NP,), F32),
            pltpu.VMEM((NS, STRIPE), F32),
            pltpu.VMEM_SHARED((NS, NP), F32),
        ],
    )(dst2d)


def _sc_keepsum(src2d, dst2d, keep1d):
    def body(src2d_r, dst2d_r, keep_r, out_r, sidx, didx, lkeep, lhist, tbuf,
             slots):
        c = lax.axis_index("c")
        s = lax.axis_index("s")
        wid = s * NC + c
        pltpu.sync_copy(src2d_r.at[pl.ds(wid * CPW, CPW)], sidx)
        pltpu.sync_copy(dst2d_r.at[pl.ds(wid * CPW, CPW)], didx)
        pltpu.sync_copy(keep_r, lkeep)
        _zero_vec(lhist, NP)
        for j in range(CPW):
            for k in range(8):
                siv = sidx[j, pl.ds(k * 16, 16)]
                kv = plsc.load_gather(lkeep, [siv])
                div = didx[j, pl.ds(k * 16, 16)]
                plsc.addupdate_scatter(lhist, [div], kv)
        _merge_tile_hists(s, c, lhist, slots, tbuf, out_r)

    return pl.kernel(
        body,
        out_type=jax.ShapeDtypeStruct((NC, NP), F32),
        mesh=_sc_mesh(),
        compiler_params=pltpu.CompilerParams(needs_layout_passes=False),
        scratch_types=[
            pltpu.VMEM((CPW, CHUNK), I32),
            pltpu.VMEM((CPW, CHUNK), I32),
            pltpu.VMEM((NP,), F32),
            pltpu.VMEM((NP,), F32),
            pltpu.VMEM((NS, STRIPE), F32),
            pltpu.VMEM_SHARED((NS, NP), F32),
        ],
    )(src2d, dst2d, keep1d)


def _tc_pre_body(x_ref, w1_ref, b1_ref, hist_ref, y1_ref, self1_ref, dis1_ref):
    xw = jnp.dot(x_ref[...], w1_ref[...], preferred_element_type=F32)
    indeg = hist_ref[0] + hist_ref[1]
    dis = lax.rsqrt(indeg + 1.0)
    y1_ref[...] = xw * dis
    self1_ref[...] = xw * (dis * dis) + b1_ref[...]
    dis1_ref[...] = dis


def _tc_pre(x, W1, b1row, hist_col):
    return pl.pallas_call(
        _tc_pre_body,
        out_shape=(
            jax.ShapeDtypeStruct((N, H), F32),
            jax.ShapeDtypeStruct((N, H), F32),
            jax.ShapeDtypeStruct((N, 1), F32),
        ),
    )(x, W1, b1row, hist_col)


def _tc_h_body(agg_ref, dis1_ref, self1_ref, p_ref, h_ref, z_ref):
    aggsum = agg_ref[0, :N, :] + agg_ref[1, :N, :]
    dis = dis1_ref[...]
    h = jnp.maximum(aggsum * dis + self1_ref[...], 0.0)
    h_ref[...] = h
    p = p_ref[...]
    nrm = jnp.sqrt(jnp.sum(p * p))
    z_ref[...] = jnp.tanh(jnp.dot(h, p, preferred_element_type=F32) / nrm)


def _tc_h(agg1, dis1_col, self1, p_col):
    return pl.pallas_call(
        _tc_h_body,
        out_shape=(
            jax.ShapeDtypeStruct((N, H), F32),
            jax.ShapeDtypeStruct((N, 1), F32),
        ),
    )(agg1, dis1_col, self1, p_col)


def _tc_topk_body(z_ref, batch_ref, keep_ref, gate_ref):
    score = z_ref[...]
    score = jnp.where(score == 0.0, 0.0, score)
    bits = lax.bitcast_convert_type(score, I32)
    key = bits ^ ((bits >> 31) & jnp.int32(0x7FFFFFFF))
    batch = batch_ref[...]

    masks = [batch == g for g in range(G)]
    cnt = [jnp.sum(jnp.where(masks[g], 1.0, 0.0)) for g in range(G)]
    kf = [jnp.floor((cnt[g] + 1.0) * 0.5) for g in range(G)]

    lo0 = jnp.int32(-1065353218)
    hi0 = jnp.int32(1065353217)

    def body(_, carry):
        los, his = carry[:G], carry[G:]
        nlo, nhi = [], []
        for g in range(G):
            lo, hi = los[g], his[g]
            mid = lo + (hi - lo + 1) // 2
            cg = jnp.sum(jnp.where(masks[g] & (key >= mid), 1.0, 0.0))
            ok = cg >= kf[g]
            nlo.append(jnp.where(ok, mid, lo))
            nhi.append(jnp.where(ok, hi, mid - jnp.int32(1)))
        return tuple(nlo) + tuple(nhi)

    init = tuple([lo0] * G) + tuple([hi0] * G)
    res = lax.fori_loop(0, 32, body, init)
    v = res[:G]

    vbc = jnp.full(key.shape, jnp.int32(-2147483648))
    for g in range(G):
        vbc = jnp.where(masks[g], v[g], vbc)
    gt = key > vbc
    tie = key == vbc

    needbc = jnp.full(key.shape, -1.0)
    sbc = jnp.zeros(key.shape, F32)
    s_run = jnp.float32(0.0)
    for g in range(G):
        cnt_gt = jnp.sum(jnp.where(masks[g] & gt, 1.0, 0.0))
        needbc = jnp.where(masks[g], kf[g] - cnt_gt, needbc)
        sbc = jnp.where(masks[g], s_run, sbc)
        s_run = s_run + jnp.sum(jnp.where(masks[g] & tie, 1.0, 0.0))

    tie_f = jnp.where(tie, 1.0, 0.0)
    r1 = lax.broadcasted_iota(I32, (128, 128), 0)
    c1 = lax.broadcasted_iota(I32, (128, 128), 1)
    u_incl = jnp.where(r1 <= c1, 1.0, 0.0)
    pc = jnp.dot(tie_f, u_incl, preferred_element_type=F32)
    rt = jnp.dot(tie_f, jnp.ones((128, 1), F32), preferred_element_type=F32)
    r2 = lax.broadcasted_iota(I32, (ROWS, ROWS), 0)
    c2 = lax.broadcasted_iota(I32, (ROWS, ROWS), 1)
    l_strict = jnp.where(r2 > c2, 1.0, 0.0)
    row_off = jnp.dot(l_strict, rt, preferred_element_type=F32)
    excl = pc - tie_f + row_off
    tie_rank = excl - sbc

    keep = jnp.where(gt | (tie & (tie_rank < needbc)), 1.0, 0.0)
    keep_ref[...] = keep
    gate_ref[...] = z_ref[...] * keep


def _tc_topk(z2d, batch2d):
    return pl.pallas_call(
        _tc_topk_body,
        out_shape=(
            jax.ShapeDtypeStruct((ROWS, 128), F32),
            jax.ShapeDtypeStruct((ROWS, 128), F32),
        ),
    )(z2d, batch2d)


def _tc_mid_body(h_ref, gate_ref, keep_ref, w2_ref, b2_ref, ks_ref,
                 y2_ref, self2_ref, d2k_ref):
    xw2 = jnp.dot(h_ref[...] * gate_ref[...], w2_ref[...],
                  preferred_element_type=F32)
    keep = keep_ref[...]
    ks = ks_ref[0] + ks_ref[1]
    deg2 = keep * (ks + 1.0)
    deg2 = jnp.where(deg2 > 0.0, deg2, 1.0)
    dis2 = lax.rsqrt(deg2)
    y2_ref[...] = xw2 * dis2
    self2_ref[...] = xw2 * (dis2 * dis2) + b2_ref[...] * keep
    d2k_ref[...] = dis2 * keep


def _tc_mid(h, gate_col, keep_col, W2, b2row, ks_col):
    return pl.pallas_call(
        _tc_mid_body,
        out_shape=(
            jax.ShapeDtypeStruct((N, H), F32),
            jax.ShapeDtypeStruct((N, H), F32),
            jax.ShapeDtypeStruct((N, 1), F32),
        ),
    )(h, gate_col, keep_col, W2, b2row, ks_col)


def _tc_post_body(agg_ref, d2k_ref, self2_ref, keep_ref, batch_ref, lw_ref,
                  lb_ref, out_ref):
    aggsum = agg_ref[0, :N, :] + agg_ref[1, :N, :]
    h2 = jnp.maximum(aggsum * d2k_ref[...] + self2_ref[...], 0.0)
    iota_g = lax.broadcasted_iota(I32, (G, N), 0)
    oh = jnp.where(iota_g == batch_ref[...], 1.0, 0.0)
    summ = jnp.dot(oh, h2, preferred_element_type=F32)
    cnt = jnp.dot(oh, keep_ref[...], preferred_element_type=F32)
    mean = summ / jnp.maximum(cnt, 1.0)
    out_ref[...] = jnp.dot(mean, lw_ref[...], preferred_element_type=F32) \
        + lb_ref[...]


def _tc_post(agg2, d2k_col, self2, keep_col, batch_row, linW, linbrow):
    return pl.pallas_call(
        _tc_post_body,
        out_shape=jax.ShapeDtypeStruct((G, C), F32),
    )(agg2, d2k_col, self2, keep_col, batch_row, linW, linbrow)


def kernel(x, edge_index, batch, W1, b1, p, W2, b2, linW, linb):
    padlen = PADC * CHUNK - E
    pad_idx = (N + (jnp.arange(padlen) % (NP - N))).astype(I32)
    src2d = jnp.concatenate([edge_index[0], pad_idx]).reshape(PADC, CHUNK)
    dst2d = jnp.concatenate([edge_index[1], pad_idx]).reshape(PADC, CHUNK)
    spadlen = SNCH * SCH - E
    spad_s = (jnp.arange(spadlen) % 240).astype(I32)
    spad_d = (N + (jnp.arange(spadlen) % (NP - N))).astype(I32)
    srcs = jnp.concatenate([edge_index[0], spad_s]).reshape(SNCH, SCH)
    dsts = jnp.concatenate([edge_index[1], spad_d]).reshape(SNCH, SCH)
    batch_p = jnp.pad(batch, (0, NP - N), constant_values=G)
    batch2d = batch_p.reshape(ROWS, 128)
    batch_row = batch.reshape(1, N)

    hist = _sc_degree(dst2d)
    hist_col = hist.reshape(NC, NP, 1)[:, :N]
    y1, self1, dis1_col = _tc_pre(x, W1, b1.reshape(1, H), hist_col)

    agg1 = _sc_spmm(srcs, dsts, y1)
    h, z_col = _tc_h(agg1, dis1_col, self1, p.reshape(H, 1))

    z2d = jnp.pad(z_col.reshape(N), (0, NP - N)).reshape(ROWS, 128)
    keep2d, gate2d = _tc_topk(z2d, batch2d)
    keep1d = keep2d.reshape(NP)
    keep_col = keep1d[:N].reshape(N, 1)
    gate_col = gate2d.reshape(NP)[:N].reshape(N, 1)

    ks = _sc_keepsum(src2d, dst2d, keep1d)
    ks_col = ks.reshape(NC, NP, 1)[:, :N]
    y2, self2, d2k_col = _tc_mid(h, gate_col, keep_col, W2,
                                 b2.reshape(1, H), ks_col)

    agg2 = _sc_spmm(srcs, dsts, y2)
    return _tc_post(agg2, d2k_col, self2, keep_col, batch_row, linW,
                    linb.reshape(1, C))


# speedup vs baseline: 2.1684x; 1.0012x over previous
import jax
import jax.numpy as jnp
from jax import lax
from jax.experimental import pallas as pl
from jax.experimental.pallas import tpu as pltpu
from jax.experimental.pallas import tpu_sc as plsc

N = 10000
E = 320000
H = 128
C = 10
G = 16
NP = 10240
ROWS = NP // 128
NC, NS = 2, 16
NW = NC * NS
CHUNK = 128
CPW = 80
PADC = NW * CPW
STRIPE = NP // NS
SCH = 100
SCPW = 104
SNCH = NW * SCPW
F32 = jnp.float32
I32 = jnp.int32


def _sc_mesh():
    return plsc.VectorSubcoreMesh(
        core_axis_name="c", subcore_axis_name="s", num_cores=NC, num_subcores=NS
    )


def _spmm_body(src2d, dst2d, y_hbm, zeros_hbm, out_hbm, sidx, didx, rows, acc,
               gsem, ssem, isem):
    c = lax.axis_index("c")
    s = lax.axis_index("s")
    wid = s * NC + c
    base = wid * SCPW
    nblk = SCPW // 8
    pltpu.sync_copy(src2d.at[pl.ds(base, 8)], sidx.at[0])
    pltpu.sync_copy(dst2d.at[pl.ds(base, 8)], didx.at[0])
    pltpu.sync_copy(zeros_hbm, acc.at[pl.ds(s * STRIPE, STRIPE)])
    plsc.subcore_barrier()

    cps_g = [None, None, None]
    cps_s = [None, None, None]
    ipf = [None, None, None, None]
    cps_g[0] = pltpu.async_copy(y_hbm.at[sidx.at[0, 0]], rows.at[0], gsem)
    cps_g[1] = pltpu.async_copy(y_hbm.at[sidx.at[0, 1]], rows.at[1], gsem)
    for j in range(SCPW):
        q = j % 3
        qn = (j + 2) % 3
        blk = j >> 3
        off = j & 7
        if off == 0:
            if blk > 0:
                ipf[2].wait()
                ipf[3].wait()
            if blk + 1 < nblk:
                nslot = (blk + 1) & 1
                ipf[0] = pltpu.async_copy(
                    src2d.at[pl.ds(base + (blk + 1) * 8, 8)],
                    sidx.at[nslot], isem)
                ipf[1] = pltpu.async_copy(
                    dst2d.at[pl.ds(base + (blk + 1) * 8, 8)],
                    didx.at[nslot], isem)
        if off == 7:
            ipf[2], ipf[3] = ipf[0], ipf[1]
        if j >= 1:
            cps_s[qn].wait()
        if j + 2 < SCPW:
            j2 = j + 2
            cps_g[qn] = pltpu.async_copy(
                y_hbm.at[sidx.at[(j2 >> 3) & 1, j2 & 7]], rows.at[qn], gsem)
        cps_g[q].wait()
        cps_s[q] = pltpu.async_copy(
            rows.at[q], acc.at[didx.at[blk & 1, off]], ssem, add=True)
    cps_s[(SCPW - 1) % 3].wait()

    plsc.subcore_barrier()
    pltpu.sync_copy(acc.at[pl.ds(s * STRIPE, STRIPE)],
                    out_hbm.at[c, pl.ds(s * STRIPE, STRIPE)])


def _sc_spmm(src2d, dst2d, y):
    zeros = jnp.zeros((STRIPE, 128), F32)
    return pl.kernel(
        _spmm_body,
        out_type=jax.ShapeDtypeStruct((NC, NP, 128), F32),
        mesh=_sc_mesh(),
        scratch_types=[
            pltpu.VMEM((2, 8, SCH), I32),
            pltpu.VMEM((2, 8, SCH), I32),
            pltpu.VMEM((3, SCH, 128), F32),
            pltpu.VMEM_SHARED((NP, 128), F32),
            pltpu.SemaphoreType.DMA,
            pltpu.SemaphoreType.DMA,
            pltpu.SemaphoreType.DMA,
        ],
    )(src2d, dst2d, y, zeros)


def _merge_tile_hists(s, c, lhist, slots, tbuf, out_r):
    pltpu.sync_copy(lhist, slots.at[s])
    plsc.subcore_barrier()
    for t in range(NS):
        pltpu.sync_copy(slots.at[t, pl.ds(s * STRIPE, STRIPE)], tbuf.at[t])

    def red(i, _):
        acc = tbuf[0, pl.ds(i * 16, 16)]
        for t in range(1, NS):
            acc = acc + tbuf[t, pl.ds(i * 16, 16)]
        lhist[pl.ds(i * 16, 16)] = acc
        return 0

    lax.fori_loop(0, STRIPE // 16, red, 0)
    pltpu.sync_copy(lhist.at[pl.ds(0, STRIPE)],
                    out_r.at[c, pl.ds(s * STRIPE, STRIPE)])


def _zero_vec(ref, nwords):
    def z(i, _):
        ref[pl.ds(i * 16, 16)] = jnp.zeros((16,), F32)
        return 0

    lax.fori_loop(0, nwords // 16, z, 0)


def _sc_degree(dst2d):
    def body(dst2d_r, out_r, didx, lhist, tbuf, slots):
        c = lax.axis_index("c")
        s = lax.axis_index("s")
        wid = s * NC + c
        pltpu.sync_copy(dst2d_r.at[pl.ds(wid * CPW, CPW)], didx)
        _zero_vec(lhist, NP)
        ones16 = jnp.ones((16,), F32)
        for j in range(CPW):
            for k in range(8):
                iv = didx[j, pl.ds(k * 16, 16)]
                plsc.addupdate_scatter(lhist, [iv], ones16)
        _merge_tile_hists(s, c, lhist, slots, tbuf, out_r)

    return pl.kernel(
        body,
        out_type=jax.ShapeDtypeStruct((NC, NP), F32),
        mesh=_sc_mesh(),
        compiler_params=pltpu.CompilerParams(needs_layout_passes=False),
        scratch_types=[
            pltpu.VMEM((CPW, CHUNK), I32),
            pltpu.VMEM((NP,), F32),
            pltpu.VMEM((NS, STRIPE), F32),
            pltpu.VMEM_SHARED((NS, NP), F32),
        ],
    )(dst2d)


def _sc_keepsum(src2d, dst2d, keep1d):
    def body(src2d_r, dst2d_r, keep_r, out_r, sidx, didx, lkeep, lhist, tbuf,
             slots):
        c = lax.axis_index("c")
        s = lax.axis_index("s")
        wid = s * NC + c
        pltpu.sync_copy(src2d_r.at[pl.ds(wid * CPW, CPW)], sidx)
        pltpu.sync_copy(dst2d_r.at[pl.ds(wid * CPW, CPW)], didx)
        pltpu.sync_copy(keep_r, lkeep)
        _zero_vec(lhist, NP)
        for j in range(CPW):
            for k in range(8):
                siv = sidx[j, pl.ds(k * 16, 16)]
                kv = plsc.load_gather(lkeep, [siv])
                div = didx[j, pl.ds(k * 16, 16)]
                plsc.addupdate_scatter(lhist, [div], kv)
        _merge_tile_hists(s, c, lhist, slots, tbuf, out_r)

    return pl.kernel(
        body,
        out_type=jax.ShapeDtypeStruct((NC, NP), F32),
        mesh=_sc_mesh(),
        compiler_params=pltpu.CompilerParams(needs_layout_passes=False),
        scratch_types=[
            pltpu.VMEM((CPW, CHUNK), I32),
            pltpu.VMEM((CPW, CHUNK), I32),
            pltpu.VMEM((NP,), F32),
            pltpu.VMEM((NP,), F32),
            pltpu.VMEM((NS, STRIPE), F32),
            pltpu.VMEM_SHARED((NS, NP), F32),
        ],
    )(src2d, dst2d, keep1d)


def _tc_pre_body(x_ref, w1_ref, b1_ref, hist_ref, y1_ref, self1_ref, dis1_ref):
    xw = jnp.dot(x_ref[...], w1_ref[...], preferred_element_type=F32)
    indeg = hist_ref[0] + hist_ref[1]
    dis = lax.rsqrt(indeg + 1.0)
    y1_ref[...] = xw * dis
    self1_ref[...] = xw * (dis * dis) + b1_ref[...]
    dis1_ref[...] = dis


def _tc_pre(x, W1, b1row, hist_col):
    return pl.pallas_call(
        _tc_pre_body,
        out_shape=(
            jax.ShapeDtypeStruct((N, H), F32),
            jax.ShapeDtypeStruct((N, H), F32),
            jax.ShapeDtypeStruct((N, 1), F32),
        ),
    )(x, W1, b1row, hist_col)


def _tc_h_body(agg_ref, dis1_ref, self1_ref, p_ref, h_ref, z_ref):
    aggsum = agg_ref[0, :N, :] + agg_ref[1, :N, :]
    dis = dis1_ref[...]
    h = jnp.maximum(aggsum * dis + self1_ref[...], 0.0)
    h_ref[...] = h
    p = p_ref[...]
    nrm = jnp.sqrt(jnp.sum(p * p))
    z_ref[...] = jnp.tanh(jnp.dot(h, p, preferred_element_type=F32) / nrm)


def _tc_h(agg1, dis1_col, self1, p_col):
    return pl.pallas_call(
        _tc_h_body,
        out_shape=(
            jax.ShapeDtypeStruct((N, H), F32),
            jax.ShapeDtypeStruct((N, 1), F32),
        ),
    )(agg1, dis1_col, self1, p_col)


def _tc_topk_body(z_ref, batch_ref, keep_ref, gate_ref):
    score = z_ref[...]
    score = jnp.where(score == 0.0, 0.0, score)
    bits = lax.bitcast_convert_type(score, I32)
    key = bits ^ ((bits >> 31) & jnp.int32(0x7FFFFFFF))
    batch = batch_ref[...]

    masks = [batch == g for g in range(G)]
    cnt = [jnp.sum(jnp.where(masks[g], 1.0, 0.0)) for g in range(G)]
    kf = [jnp.floor((cnt[g] + 1.0) * 0.5) for g in range(G)]

    lo0 = jnp.int32(-1065353218)
    hi0 = jnp.int32(1065353217)

    def body(_, carry):
        los, his = carry[:G], carry[G:]
        nlo, nhi = [], []
        for g in range(G):
            lo, hi = los[g], his[g]
            mid = lo + (hi - lo + 1) // 2
            cg = jnp.sum(jnp.where(masks[g] & (key >= mid), 1.0, 0.0))
            ok = cg >= kf[g]
            nlo.append(jnp.where(ok, mid, lo))
            nhi.append(jnp.where(ok, hi, mid - jnp.int32(1)))
        return tuple(nlo) + tuple(nhi)

    init = tuple([lo0] * G) + tuple([hi0] * G)
    res = lax.fori_loop(0, 31, body, init)
    v = res[:G]

    vbc = jnp.full(key.shape, jnp.int32(-2147483648))
    for g in range(G):
        vbc = jnp.where(masks[g], v[g], vbc)
    gt = key > vbc
    tie = key == vbc

    needbc = jnp.full(key.shape, -1.0)
    sbc = jnp.zeros(key.shape, F32)
    s_run = jnp.float32(0.0)
    for g in range(G):
        cnt_gt = jnp.sum(jnp.where(masks[g] & gt, 1.0, 0.0))
        needbc = jnp.where(masks[g], kf[g] - cnt_gt, needbc)
        sbc = jnp.where(masks[g], s_run, sbc)
        s_run = s_run + jnp.sum(jnp.where(masks[g] & tie, 1.0, 0.0))

    tie_f = jnp.where(tie, 1.0, 0.0)
    r1 = lax.broadcasted_iota(I32, (128, 128), 0)
    c1 = lax.broadcasted_iota(I32, (128, 128), 1)
    u_incl = jnp.where(r1 <= c1, 1.0, 0.0)
    pc = jnp.dot(tie_f, u_incl, preferred_element_type=F32)
    rt = jnp.dot(tie_f, jnp.ones((128, 1), F32), preferred_element_type=F32)
    r2 = lax.broadcasted_iota(I32, (ROWS, ROWS), 0)
    c2 = lax.broadcasted_iota(I32, (ROWS, ROWS), 1)
    l_strict = jnp.where(r2 > c2, 1.0, 0.0)
    row_off = jnp.dot(l_strict, rt, preferred_element_type=F32)
    excl = pc - tie_f + row_off
    tie_rank = excl - sbc

    keep = jnp.where(gt | (tie & (tie_rank < needbc)), 1.0, 0.0)
    keep_ref[...] = keep
    gate_ref[...] = z_ref[...] * keep


def _tc_topk(z2d, batch2d):
    return pl.pallas_call(
        _tc_topk_body,
        out_shape=(
            jax.ShapeDtypeStruct((ROWS, 128), F32),
            jax.ShapeDtypeStruct((ROWS, 128), F32),
        ),
    )(z2d, batch2d)


def _tc_mid_body(h_ref, gate_ref, keep_ref, w2_ref, b2_ref, ks_ref,
                 y2_ref, self2_ref, d2k_ref):
    xw2 = jnp.dot(h_ref[...] * gate_ref[...], w2_ref[...],
                  preferred_element_type=F32)
    keep = keep_ref[...]
    ks = ks_ref[0] + ks_ref[1]
    deg2 = keep * (ks + 1.0)
    deg2 = jnp.where(deg2 > 0.0, deg2, 1.0)
    dis2 = lax.rsqrt(deg2)
    y2_ref[...] = xw2 * dis2
    self2_ref[...] = xw2 * (dis2 * dis2) + b2_ref[...] * keep
    d2k_ref[...] = dis2 * keep


def _tc_mid(h, gate_col, keep_col, W2, b2row, ks_col):
    return pl.pallas_call(
        _tc_mid_body,
        out_shape=(
            jax.ShapeDtypeStruct((N, H), F32),
            jax.ShapeDtypeStruct((N, H), F32),
            jax.ShapeDtypeStruct((N, 1), F32),
        ),
    )(h, gate_col, keep_col, W2, b2row, ks_col)


def _tc_post_body(agg_ref, d2k_ref, self2_ref, keep_ref, batch_ref, lw_ref,
                  lb_ref, out_ref):
    aggsum = agg_ref[0, :N, :] + agg_ref[1, :N, :]
    h2 = jnp.maximum(aggsum * d2k_ref[...] + self2_ref[...], 0.0)
    iota_g = lax.broadcasted_iota(I32, (G, N), 0)
    oh = jnp.where(iota_g == batch_ref[...], 1.0, 0.0)
    summ = jnp.dot(oh, h2, preferred_element_type=F32)
    cnt = jnp.dot(oh, keep_ref[...], preferred_element_type=F32)
    mean = summ / jnp.maximum(cnt, 1.0)
    out_ref[...] = jnp.dot(mean, lw_ref[...], preferred_element_type=F32) \
        + lb_ref[...]


def _tc_post(agg2, d2k_col, self2, keep_col, batch_row, linW, linbrow):
    return pl.pallas_call(
        _tc_post_body,
        out_shape=jax.ShapeDtypeStruct((G, C), F32),
    )(agg2, d2k_col, self2, keep_col, batch_row, linW, linbrow)


def kernel(x, edge_index, batch, W1, b1, p, W2, b2, linW, linb):
    padlen = PADC * CHUNK - E
    pad_idx = (N + (jnp.arange(padlen) % (NP - N))).astype(I32)
    src2d = jnp.concatenate([edge_index[0], pad_idx]).reshape(PADC, CHUNK)
    dst2d = jnp.concatenate([edge_index[1], pad_idx]).reshape(PADC, CHUNK)
    spadlen = SNCH * SCH - E
    spad_s = (jnp.arange(spadlen) % 240).astype(I32)
    spad_d = (N + (jnp.arange(spadlen) % (NP - N))).astype(I32)
    srcs = jnp.concatenate([edge_index[0], spad_s]).reshape(SNCH, SCH)
    dsts = jnp.concatenate([edge_index[1], spad_d]).reshape(SNCH, SCH)
    batch_p = jnp.pad(batch, (0, NP - N), constant_values=G)
    batch2d = batch_p.reshape(ROWS, 128)
    batch_row = batch.reshape(1, N)

    hist = _sc_degree(dst2d)
    hist_col = hist.reshape(NC, NP, 1)[:, :N]
    y1, self1, dis1_col = _tc_pre(x, W1, b1.reshape(1, H), hist_col)

    agg1 = _sc_spmm(srcs, dsts, y1)
    h, z_col = _tc_h(agg1, dis1_col, self1, p.reshape(H, 1))

    z2d = jnp.pad(z_col.reshape(N), (0, NP - N)).reshape(ROWS, 128)
    keep2d, gate2d = _tc_topk(z2d, batch2d)
    keep1d = keep2d.reshape(NP)
    keep_col = keep1d[:N].reshape(N, 1)
    gate_col = gate2d.reshape(NP)[:N].reshape(N, 1)

    ks = _sc_keepsum(src2d, dst2d, keep1d)
    ks_col = ks.reshape(NC, NP, 1)[:, :N]
    y2, self2, d2k_col = _tc_mid(h, gate_col, keep_col, W2,
                                 b2.reshape(1, H), ks_col)

    agg2 = _sc_spmm(srcs, dsts, y2)
    return _tc_post(agg2, d2k_col, self2, keep_col, batch_row, linW,
                    linb.reshape(1, C))
